# initial kernel scaffold (unmeasured)
import jax
import jax.numpy as jnp
from jax import lax
from jax.experimental import pallas as pl
from jax.experimental.pallas import tpu as pltpu

N_DEV = 4
SCALE = 0.08838834764831843
HQ, HKV, DH = 8, 2, 128
GQA = HQ // HKV


def kernel(x, Wq, Wo, K_ext, V_ext):
    _, sq, d = x.shape
    _, skv, _, _ = K_ext.shape

    def body(x_ref, wq_ref, wo_ref, k_ref, v_ref, out_ref,
             comm_o, comm_s, so_sems, ro_sems, ss_sems, rs_sems):
        my = lax.axis_index("i")
        left = (my - 1) % N_DEV
        right = (my + 1) % N_DEV

        barrier = pltpu.get_barrier_semaphore()
        for nbr in (left, right):
            pl.semaphore_signal(barrier, inc=1, device_id=(nbr,),
                                device_id_type=pl.DeviceIdType.MESH)
        pl.semaphore_wait(barrier, 2)

        q = jnp.dot(x_ref[0], wq_ref[...],
                    preferred_element_type=jnp.float32)
        q = q.reshape(sq, HQ, DH) * SCALE

        o_heads, m_heads, l_heads = [], [], []
        for h in range(HQ):
            kh = k_ref[0, :, h // GQA, :]
            vh = v_ref[0, :, h // GQA, :]
            s = lax.dot_general(q[:, h, :], kh,
                                (((1,), (1,)), ((), ())),
                                preferred_element_type=jnp.float32)
            mh = jnp.max(s, axis=1)
            p = jnp.exp(s - mh[:, None])
            lh = jnp.sum(p, axis=1)
            oh = jnp.dot(p, vh, preferred_element_type=jnp.float32)
            o_heads.append(oh)
            m_heads.append(mh)
            l_heads.append(lh)

        comm_o[0] = jnp.stack(o_heads, axis=1)
        comm_s[0, 0] = jnp.stack(m_heads, axis=1)
        comm_s[0, 1] = jnp.stack(l_heads, axis=1)

        rd_o, rd_s = [], []
        for h in range(N_DEV - 1):
            rd_o.append(pltpu.make_async_remote_copy(
                src_ref=comm_o.at[h], dst_ref=comm_o.at[h + 1],
                send_sem=so_sems.at[h], recv_sem=ro_sems.at[h + 1],
                device_id=(right,), device_id_type=pl.DeviceIdType.MESH))
            rd_s.append(pltpu.make_async_remote_copy(
                src_ref=comm_s.at[h], dst_ref=comm_s.at[h + 1],
                send_sem=ss_sems.at[h], recv_sem=rs_sems.at[h + 1],
                device_id=(right,), device_id_type=pl.DeviceIdType.MESH))

        rd_o[0].start()
        rd_s[0].start()
        for h in range(N_DEV - 1):
            rd_o[h].wait_recv()
            rd_s[h].wait_recv()
            if h + 1 < N_DEV - 1:
                rd_o[h + 1].start()
                rd_s[h + 1].start()

        co = comm_o[...]
        cs = comm_s[...]
        m_all, l_all = cs[:, 0], cs[:, 1]
        m_new = jnp.max(m_all, axis=0)
        w = jnp.exp(m_all - m_new[None])
        o_merged = jnp.sum(co * w[..., None], axis=0)
        l_merged = jnp.sum(l_all * w, axis=0)

        attn = (o_merged / l_merged[..., None]).reshape(sq, HQ * DH)
        out_ref[0] = jnp.dot(attn, wo_ref[...],
                             preferred_element_type=jnp.float32)

        for h in range(N_DEV - 1):
            rd_o[h].wait_send()
            rd_s[h].wait_send()

    return pl.pallas_call(
        body,
        out_shape=jax.ShapeDtypeStruct((1, sq, d), jnp.float32),
        in_specs=[pl.BlockSpec(memory_space=pltpu.VMEM)] * 5,
        out_specs=pl.BlockSpec(memory_space=pltpu.VMEM),
        scratch_shapes=[
            pltpu.VMEM((N_DEV, sq, HQ, DH), jnp.float32),
            pltpu.VMEM((N_DEV, 2, sq, HQ), jnp.float32),
            pltpu.SemaphoreType.DMA((N_DEV,)),
            pltpu.SemaphoreType.DMA((N_DEV,)),
            pltpu.SemaphoreType.DMA((N_DEV,)),
            pltpu.SemaphoreType.DMA((N_DEV,)),
        ],
        compiler_params=pltpu.CompilerParams(collective_id=0),
    )(x, Wq, Wo, K_ext, V_ext)


# baseline (device time: 123260 ns/iter reference)
import jax
import jax.numpy as jnp
from jax import lax
from jax.experimental import pallas as pl
from jax.experimental.pallas import tpu as pltpu

N_DEV = 4
SCALE = 0.08838834764831843
HQ, HKV, DH = 8, 2, 128
GQA = HQ // HKV


def kernel(x, Wq, Wo, K_ext, V_ext):
    _, sq, d = x.shape
    _, skv, _, _ = K_ext.shape

    def body(x_ref, wq_ref, wo_ref, k_ref, v_ref, out_ref,
             comm_o, comm_s, so_sems, ro_sems, ss_sems, rs_sems):
        my = lax.axis_index("i")
        left = (my - 1) % N_DEV
        right = (my + 1) % N_DEV

        barrier = pltpu.get_barrier_semaphore()
        for nbr in (left, right):
            pl.semaphore_signal(barrier, inc=1, device_id=(nbr,),
                                device_id_type=pl.DeviceIdType.MESH)
        pl.semaphore_wait(barrier, 2)

        q = jnp.dot(x_ref[0], wq_ref[...],
                    preferred_element_type=jnp.float32)
        q = q.reshape(sq, HQ, DH) * SCALE

        o_heads, m_heads, l_heads = [], [], []
        for h in range(HQ):
            kh = k_ref[0, :, h // GQA, :]
            vh = v_ref[0, :, h // GQA, :]
            s = lax.dot_general(q[:, h, :], kh,
                                (((1,), (1,)), ((), ())),
                                preferred_element_type=jnp.float32)
            mh = jnp.max(s, axis=1)
            p = jnp.exp(s - mh[:, None])
            lh = jnp.sum(p, axis=1)
            oh = jnp.dot(p, vh, preferred_element_type=jnp.float32)
            o_heads.append(oh)
            m_heads.append(mh)
            l_heads.append(lh)

        comm_o[0] = jnp.stack(o_heads, axis=1)
        comm_s[0, 0] = jnp.stack(m_heads, axis=1)
        comm_s[0, 1] = jnp.stack(l_heads, axis=1)

        rd_o, rd_s = [], []
        for h in range(N_DEV - 1):
            rd_o.append(pltpu.make_async_remote_copy(
                src_ref=comm_o.at[h], dst_ref=comm_o.at[h + 1],
                send_sem=so_sems.at[h], recv_sem=ro_sems.at[h + 1],
                device_id=(right,), device_id_type=pl.DeviceIdType.MESH))
            rd_s.append(pltpu.make_async_remote_copy(
                src_ref=comm_s.at[h], dst_ref=comm_s.at[h + 1],
                send_sem=ss_sems.at[h], recv_sem=rs_sems.at[h + 1],
                device_id=(right,), device_id_type=pl.DeviceIdType.MESH))

        rd_o[0].start()
        rd_s[0].start()
        for h in range(N_DEV - 1):
            rd_o[h].wait_recv()
            rd_s[h].wait_recv()
            if h + 1 < N_DEV - 1:
                rd_o[h + 1].start()
                rd_s[h + 1].start()

        co = comm_o[...]
        cs = comm_s[...]
        m_all, l_all = cs[:, 0], cs[:, 1]
        m_new = jnp.max(m_all, axis=0)
        w = jnp.exp(m_all - m_new[None])
        o_merged = jnp.sum(co * w[..., None], axis=0)
        l_merged = jnp.sum(l_all * w, axis=0)

        attn = (o_merged / l_merged[..., None]).reshape(sq, HQ * DH)
        out_ref[0] = jnp.dot(attn, wo_ref[...],
                             preferred_element_type=jnp.float32)

        for h in range(N_DEV - 1):
            rd_o[h].wait_send()
            rd_s[h].wait_send()

    return pl.pallas_call(
        body,
        out_shape=jax.ShapeDtypeStruct((1, sq, d), jnp.float32),
        in_specs=[pl.BlockSpec(memory_space=pltpu.VMEM)] * 5,
        out_specs=pl.BlockSpec(memory_space=pltpu.VMEM),
        scratch_shapes=[
            pltpu.VMEM((N_DEV, sq, HQ, DH), jnp.float32),
            pltpu.VMEM((N_DEV, 2, sq, HQ), jnp.float32),
            pltpu.SemaphoreType.DMA((N_DEV,)),
            pltpu.SemaphoreType.DMA((N_DEV,)),
            pltpu.SemaphoreType.DMA((N_DEV,)),
            pltpu.SemaphoreType.DMA((N_DEV,)),
        ],
        compiler_params=pltpu.CompilerParams(
            collective_id=0, vmem_limit_bytes=100 * 1024 * 1024
        ),
    )(x, Wq, Wo, K_ext, V_ext)


# device time: 78545 ns/iter; 1.5693x vs baseline; 1.5693x over previous
import jax
import jax.numpy as jnp
from jax import lax
from jax.experimental import pallas as pl
from jax.experimental.pallas import tpu as pltpu

N_DEV = 4
SCALE = 0.08838834764831843
HQ, HKV, DH = 8, 2, 128
GQA = HQ // HKV


def kernel(x, Wq, Wo, K_ext, V_ext):
    _, sq, d = x.shape
    half = sq // 2

    def body(x_ref, wq_ref, wo_ref, k_ref, v_ref, out_ref,
             o_cw, o_ccw, s_cw, s_ccw,
             so_cw, ro_cw, so_ccw, ro_ccw,
             ss_cw, rs_cw, ss_ccw, rs_ccw):
        my = lax.axis_index("i")
        left = (my - 1) % N_DEV
        right = (my + 1) % N_DEV

        barrier = pltpu.get_barrier_semaphore()
        for nbr in (left, right):
            pl.semaphore_signal(barrier, inc=1, device_id=(nbr,),
                                device_id_type=pl.DeviceIdType.MESH)
        pl.semaphore_wait(barrier, 2)

        q = jnp.dot(x_ref[0], wq_ref[...],
                    preferred_element_type=jnp.float32)
        q = q.reshape(sq, HQ, DH) * SCALE

        o_heads, m_heads, l_heads = [], [], []
        for h in range(HQ):
            kh = k_ref[0, :, h // GQA, :]
            vh = v_ref[0, :, h // GQA, :]
            s = lax.dot_general(q[:, h, :], kh,
                                (((1,), (1,)), ((), ())),
                                preferred_element_type=jnp.float32)
            mh = jnp.max(s, axis=1)
            p = jnp.exp(s - mh[:, None])
            lh = jnp.sum(p, axis=1)
            oh = jnp.dot(p, vh, preferred_element_type=jnp.float32)
            o_heads.append(oh)
            m_heads.append(mh)
            l_heads.append(lh)

        o_all = jnp.stack(o_heads, axis=1)
        m_all = jnp.stack(m_heads, axis=1)
        l_all = jnp.stack(l_heads, axis=1)

        o_cw[0] = o_all[:half]
        o_ccw[0] = o_all[half:]
        s_cw[0, 0], s_cw[0, 1] = m_all[:half], l_all[:half]
        s_ccw[0, 0], s_ccw[0, 1] = m_all[half:], l_all[half:]

        def make_ring(buf, sem_s, sem_r, dst):
            rds = []
            for h in range(N_DEV - 1):
                rds.append(pltpu.make_async_remote_copy(
                    src_ref=buf.at[h], dst_ref=buf.at[h + 1],
                    send_sem=sem_s.at[h], recv_sem=sem_r.at[h + 1],
                    device_id=(dst,), device_id_type=pl.DeviceIdType.MESH))
            return rds

        rings = [
            make_ring(o_cw, so_cw, ro_cw, right),
            make_ring(s_cw, ss_cw, rs_cw, right),
            make_ring(o_ccw, so_ccw, ro_ccw, left),
            make_ring(s_ccw, ss_ccw, rs_ccw, left),
        ]
        for r in rings:
            r[0].start()
        for h in range(N_DEV - 1):
            for r in rings:
                r[h].wait_recv()
                if h + 1 < N_DEV - 1:
                    r[h + 1].start()

        def merge_and_project(o_buf, s_buf, row0):
            co = o_buf[...]
            cs = s_buf[...]
            ms, ls = cs[:, 0], cs[:, 1]
            m_new = jnp.max(ms, axis=0)
            w = jnp.exp(ms - m_new[None])
            o_m = jnp.sum(co * w[..., None], axis=0)
            l_m = jnp.sum(ls * w, axis=0)
            attn = (o_m / l_m[..., None]).reshape(half, HQ * DH)
            out_ref[0, row0:row0 + half, :] = jnp.dot(
                attn, wo_ref[...], preferred_element_type=jnp.float32)

        merge_and_project(o_cw, s_cw, 0)
        merge_and_project(o_ccw, s_ccw, half)

        for r in rings:
            for h in range(N_DEV - 1):
                r[h].wait_send()

    return pl.pallas_call(
        body,
        out_shape=jax.ShapeDtypeStruct((1, sq, d), jnp.float32),
        in_specs=[pl.BlockSpec(memory_space=pltpu.VMEM)] * 5,
        out_specs=pl.BlockSpec(memory_space=pltpu.VMEM),
        scratch_shapes=[
            pltpu.VMEM((N_DEV, half, HQ, DH), jnp.float32),
            pltpu.VMEM((N_DEV, half, HQ, DH), jnp.float32),
            pltpu.VMEM((N_DEV, 2, half, HQ), jnp.float32),
            pltpu.VMEM((N_DEV, 2, half, HQ), jnp.float32),
            pltpu.SemaphoreType.DMA((N_DEV,)),
            pltpu.SemaphoreType.DMA((N_DEV,)),
            pltpu.SemaphoreType.DMA((N_DEV,)),
            pltpu.SemaphoreType.DMA((N_DEV,)),
            pltpu.SemaphoreType.DMA((N_DEV,)),
            pltpu.SemaphoreType.DMA((N_DEV,)),
            pltpu.SemaphoreType.DMA((N_DEV,)),
            pltpu.SemaphoreType.DMA((N_DEV,)),
        ],
        compiler_params=pltpu.CompilerParams(
            collective_id=0, vmem_limit_bytes=100 * 1024 * 1024
        ),
    )(x, Wq, Wo, K_ext, V_ext)
